# initial kernel scaffold (unmeasured)
import jax
import jax.numpy as jnp
from jax import lax
from jax.experimental import pallas as pl
from jax.experimental.pallas import tpu as pltpu

N_DEV = 8
M = 4096
N_TOT = 8192
CHUNK_M = M // N_DEV
HALF_N = N_TOT // 2


def kernel(x, w_mat, scale_x, scale_w):
    def body(x_ref, w_ref, sx_ref, sw_ref, out_ref,
             send_cw, comm_cw, send_ccw, comm_ccw,
             sems_cw, sems_ccw, copy_sem, credit_cw, credit_ccw):
        my = lax.axis_index("i")
        left = lax.rem(my + N_DEV - 1, N_DEV)
        right = lax.rem(my + 1, N_DEV)

        barrier = pltpu.get_barrier_semaphore()
        for nbr in (left, right):
            pl.semaphore_signal(barrier, inc=1, device_id=(nbr,),
                                device_id_type=pl.DeviceIdType.MESH)
        pl.semaphore_wait(barrier, 2)

        scale = sx_ref[0] * sw_ref[0]

        def partial(c, half):
            xa = x_ref[pl.ds(c * CHUNK_M, CHUNK_M), :]
            wa = w_ref[:, half * HALF_N:(half + 1) * HALF_N]
            acc = lax.dot_general(xa, wa, (((1,), (0,)), ((), ())),
                                  preferred_element_type=jnp.int32)
            return acc.astype(jnp.float32) * scale

        def store_out(buf_ref, c, half):
            cp = pltpu.make_async_copy(
                buf_ref,
                out_ref.at[pl.ds(c * CHUNK_M, CHUNK_M),
                           pl.ds(half * HALF_N, HALF_N)],
                copy_sem)
            cp.start()
            cp.wait()

        send_cw[...] = partial(my, 0)
        send_ccw[...] = partial(my, 1)

        for s in range(2 * N_DEV - 2):
            if s >= 1:
                pl.semaphore_wait(credit_cw, 1)
                pl.semaphore_wait(credit_ccw, 1)
            rd_cw = pltpu.make_async_remote_copy(
                src_ref=send_cw, dst_ref=comm_cw,
                send_sem=sems_cw.at[0], recv_sem=sems_cw.at[1],
                device_id=(right,), device_id_type=pl.DeviceIdType.MESH)
            rd_ccw = pltpu.make_async_remote_copy(
                src_ref=send_ccw, dst_ref=comm_ccw,
                send_sem=sems_ccw.at[0], recv_sem=sems_ccw.at[1],
                device_id=(left,), device_id_type=pl.DeviceIdType.MESH)
            rd_cw.start()
            rd_ccw.start()
            rd_cw.wait()
            rd_ccw.wait()

            if s < N_DEV - 1:
                c_cw = lax.rem(my + 2 * N_DEV - s - 1, N_DEV)
                send_cw[...] = comm_cw[...] + partial(c_cw, 0)
                pl.semaphore_signal(credit_cw, inc=1, device_id=(left,),
                                    device_id_type=pl.DeviceIdType.MESH)
                c_ccw = lax.rem(my + s + 1, N_DEV)
                send_ccw[...] = comm_ccw[...] + partial(c_ccw, 1)
                pl.semaphore_signal(credit_ccw, inc=1, device_id=(right,),
                                    device_id_type=pl.DeviceIdType.MESH)
                if s == N_DEV - 2:
                    store_out(send_cw, lax.rem(my + 1, N_DEV), 0)
                    store_out(send_ccw, lax.rem(my + N_DEV - 1, N_DEV), 1)
            else:
                t = s - (N_DEV - 1)
                store_out(comm_cw, lax.rem(my + N_DEV - t, N_DEV), 0)
                store_out(comm_ccw, lax.rem(my + t, N_DEV), 1)
                if s < 2 * N_DEV - 3:
                    send_cw[...] = comm_cw[...]
                    pl.semaphore_signal(credit_cw, inc=1, device_id=(left,),
                                        device_id_type=pl.DeviceIdType.MESH)
                    send_ccw[...] = comm_ccw[...]
                    pl.semaphore_signal(credit_ccw, inc=1, device_id=(right,),
                                        device_id_type=pl.DeviceIdType.MESH)

    return pl.pallas_call(
        body,
        out_shape=jax.ShapeDtypeStruct((M, N_TOT), jnp.float32),
        in_specs=[
            pl.BlockSpec(memory_space=pltpu.VMEM),
            pl.BlockSpec(memory_space=pltpu.VMEM),
            pl.BlockSpec(memory_space=pltpu.SMEM),
            pl.BlockSpec(memory_space=pltpu.SMEM),
        ],
        out_specs=pl.BlockSpec(memory_space=pltpu.ANY),
        scratch_shapes=[
            pltpu.VMEM((CHUNK_M, HALF_N), jnp.float32),
            pltpu.VMEM((CHUNK_M, HALF_N), jnp.float32),
            pltpu.VMEM((CHUNK_M, HALF_N), jnp.float32),
            pltpu.VMEM((CHUNK_M, HALF_N), jnp.float32),
            pltpu.SemaphoreType.DMA((2,)),
            pltpu.SemaphoreType.DMA((2,)),
            pltpu.SemaphoreType.DMA,
            pltpu.SemaphoreType.REGULAR,
            pltpu.SemaphoreType.REGULAR,
        ],
        compiler_params=pltpu.CompilerParams(collective_id=0),
    )(x, w_mat, scale_x, scale_w)


# baseline (device time: 1497355 ns/iter reference)
import jax
import jax.numpy as jnp
from jax import lax
from jax.experimental import pallas as pl
from jax.experimental.pallas import tpu as pltpu

N_DEV = 8
M = 4096
N_TOT = 8192
CHUNK_M = M // N_DEV
HALF_N = N_TOT // 2


def kernel(x, w_mat, scale_x, scale_w):
    def body(x_ref, w_ref, sx_ref, sw_ref, out_ref,
             send_cw, comm_cw, send_ccw, comm_ccw,
             sems_cw, sems_ccw, copy_sem, credit_cw, credit_ccw):
        my = lax.axis_index("i")
        left = lax.rem(my + N_DEV - 1, N_DEV)
        right = lax.rem(my + 1, N_DEV)

        barrier = pltpu.get_barrier_semaphore()
        for nbr in (left, right):
            pl.semaphore_signal(barrier, inc=1, device_id=(nbr,),
                                device_id_type=pl.DeviceIdType.MESH)
        pl.semaphore_wait(barrier, 2)

        scale = sx_ref[0] * sw_ref[0]

        def partial(c, half):
            xa = x_ref[pl.ds(c * CHUNK_M, CHUNK_M), :]
            wa = w_ref[:, half * HALF_N:(half + 1) * HALF_N]
            acc = lax.dot_general(xa, wa, (((1,), (0,)), ((), ())),
                                  preferred_element_type=jnp.int32)
            return acc.astype(jnp.float32) * scale

        def store_out(buf_ref, c, half):
            cp = pltpu.make_async_copy(
                buf_ref,
                out_ref.at[pl.ds(c * CHUNK_M, CHUNK_M),
                           pl.ds(half * HALF_N, HALF_N)],
                copy_sem)
            cp.start()
            cp.wait()

        send_cw[...] = partial(my, 0)
        send_ccw[...] = partial(my, 1)

        for s in range(2 * N_DEV - 2):
            if s >= 1:
                pl.semaphore_wait(credit_cw, 1)
                pl.semaphore_wait(credit_ccw, 1)
            rd_cw = pltpu.make_async_remote_copy(
                src_ref=send_cw, dst_ref=comm_cw,
                send_sem=sems_cw.at[0], recv_sem=sems_cw.at[1],
                device_id=(right,), device_id_type=pl.DeviceIdType.MESH)
            rd_ccw = pltpu.make_async_remote_copy(
                src_ref=send_ccw, dst_ref=comm_ccw,
                send_sem=sems_ccw.at[0], recv_sem=sems_ccw.at[1],
                device_id=(left,), device_id_type=pl.DeviceIdType.MESH)
            rd_cw.start()
            rd_ccw.start()
            rd_cw.wait()
            rd_ccw.wait()

            if s < N_DEV - 1:
                c_cw = lax.rem(my + 2 * N_DEV - s - 1, N_DEV)
                send_cw[...] = comm_cw[...] + partial(c_cw, 0)
                pl.semaphore_signal(credit_cw, inc=1, device_id=(left,),
                                    device_id_type=pl.DeviceIdType.MESH)
                c_ccw = lax.rem(my + s + 1, N_DEV)
                send_ccw[...] = comm_ccw[...] + partial(c_ccw, 1)
                pl.semaphore_signal(credit_ccw, inc=1, device_id=(right,),
                                    device_id_type=pl.DeviceIdType.MESH)
                if s == N_DEV - 2:
                    store_out(send_cw, lax.rem(my + 1, N_DEV), 0)
                    store_out(send_ccw, lax.rem(my + N_DEV - 1, N_DEV), 1)
            else:
                t = s - (N_DEV - 1)
                store_out(comm_cw, lax.rem(my + N_DEV - t, N_DEV), 0)
                store_out(comm_ccw, lax.rem(my + t, N_DEV), 1)
                if s < 2 * N_DEV - 3:
                    send_cw[...] = comm_cw[...]
                    pl.semaphore_signal(credit_cw, inc=1, device_id=(left,),
                                        device_id_type=pl.DeviceIdType.MESH)
                    send_ccw[...] = comm_ccw[...]
                    pl.semaphore_signal(credit_ccw, inc=1, device_id=(right,),
                                        device_id_type=pl.DeviceIdType.MESH)

    return pl.pallas_call(
        body,
        out_shape=jax.ShapeDtypeStruct((M, N_TOT), jnp.float32),
        in_specs=[
            pl.BlockSpec(memory_space=pltpu.VMEM),
            pl.BlockSpec(memory_space=pltpu.VMEM),
            pl.BlockSpec(memory_space=pltpu.SMEM),
            pl.BlockSpec(memory_space=pltpu.SMEM),
        ],
        out_specs=pl.BlockSpec(memory_space=pl.ANY),
        scratch_shapes=[
            pltpu.VMEM((CHUNK_M, HALF_N), jnp.float32),
            pltpu.VMEM((CHUNK_M, HALF_N), jnp.float32),
            pltpu.VMEM((CHUNK_M, HALF_N), jnp.float32),
            pltpu.VMEM((CHUNK_M, HALF_N), jnp.float32),
            pltpu.SemaphoreType.DMA((2,)),
            pltpu.SemaphoreType.DMA((2,)),
            pltpu.SemaphoreType.DMA,
            pltpu.SemaphoreType.REGULAR,
            pltpu.SemaphoreType.REGULAR,
        ],
        compiler_params=pltpu.CompilerParams(
            collective_id=0,
            vmem_limit_bytes=64 * 1024 * 1024,
        ),
    )(x, w_mat, scale_x, scale_w)


# device time: 1460283 ns/iter; 1.0254x vs baseline; 1.0254x over previous
import jax
import jax.numpy as jnp
from jax import lax
from jax.experimental import pallas as pl
from jax.experimental.pallas import tpu as pltpu

N_DEV = 8
M = 4096
N_TOT = 8192
CHUNK_M = M // N_DEV
HALF_N = N_TOT // 2


def kernel(x, w_mat, scale_x, scale_w):
    def body(x_ref, w_ref, sx_ref, sw_ref, out_ref,
             send_cw, comm_cw, send_ccw, comm_ccw, pc_cw, pc_ccw,
             sems_cw, sems_ccw, copy_sems, credit_cw, credit_ccw):
        my = lax.axis_index("i")
        left = lax.rem(my + N_DEV - 1, N_DEV)
        right = lax.rem(my + 1, N_DEV)

        barrier = pltpu.get_barrier_semaphore()
        for nbr in (left, right):
            pl.semaphore_signal(barrier, inc=1, device_id=(nbr,),
                                device_id_type=pl.DeviceIdType.MESH)
        pl.semaphore_wait(barrier, 2)

        scale = sx_ref[0] * sw_ref[0]

        def partial(c, half):
            xa = x_ref[pl.ds(c * CHUNK_M, CHUNK_M), :]
            wa = w_ref[:, half * HALF_N:(half + 1) * HALF_N]
            acc = lax.dot_general(xa, wa, (((1,), (0,)), ((), ())),
                                  preferred_element_type=jnp.int32)
            return acc.astype(jnp.float32) * scale

        def store_out(buf_ref, c, half):
            cp = pltpu.make_async_copy(
                buf_ref,
                out_ref.at[pl.ds(c * CHUNK_M, CHUNK_M),
                           pl.ds(half * HALF_N, HALF_N)],
                copy_sems.at[half])
            cp.start()
            return cp

        send_cw[...] = partial(my, 0)
        send_ccw[...] = partial(my, 1)

        for s in range(2 * N_DEV - 2):
            if s >= 1:
                pl.semaphore_wait(credit_cw, 1)
                pl.semaphore_wait(credit_ccw, 1)
            rd_cw = pltpu.make_async_remote_copy(
                src_ref=send_cw, dst_ref=comm_cw,
                send_sem=sems_cw.at[0], recv_sem=sems_cw.at[1],
                device_id=(right,), device_id_type=pl.DeviceIdType.MESH)
            rd_ccw = pltpu.make_async_remote_copy(
                src_ref=send_ccw, dst_ref=comm_ccw,
                send_sem=sems_ccw.at[0], recv_sem=sems_ccw.at[1],
                device_id=(left,), device_id_type=pl.DeviceIdType.MESH)
            rd_cw.start()
            rd_ccw.start()

            if s < N_DEV - 1:
                c_cw = lax.rem(my + 2 * N_DEV - s - 1, N_DEV)
                c_ccw = lax.rem(my + s + 1, N_DEV)
                pc_cw[...] = partial(c_cw, 0)
                pc_ccw[...] = partial(c_ccw, 1)

            rd_cw.wait()
            rd_ccw.wait()

            if s < N_DEV - 1:
                send_cw[...] = comm_cw[...] + pc_cw[...]
                pl.semaphore_signal(credit_cw, inc=1, device_id=(left,),
                                    device_id_type=pl.DeviceIdType.MESH)
                send_ccw[...] = comm_ccw[...] + pc_ccw[...]
                pl.semaphore_signal(credit_ccw, inc=1, device_id=(right,),
                                    device_id_type=pl.DeviceIdType.MESH)
                if s == N_DEV - 2:
                    cp0 = store_out(send_cw, lax.rem(my + 1, N_DEV), 0)
                    cp1 = store_out(send_ccw, lax.rem(my + N_DEV - 1, N_DEV), 1)
                    cp0.wait()
                    cp1.wait()
            else:
                t = s - (N_DEV - 1)
                cp0 = store_out(comm_cw, lax.rem(my + N_DEV - t, N_DEV), 0)
                cp1 = store_out(comm_ccw, lax.rem(my + t, N_DEV), 1)
                if s < 2 * N_DEV - 3:
                    send_cw[...] = comm_cw[...]
                    send_ccw[...] = comm_ccw[...]
                cp0.wait()
                cp1.wait()
                if s < 2 * N_DEV - 3:
                    pl.semaphore_signal(credit_cw, inc=1, device_id=(left,),
                                        device_id_type=pl.DeviceIdType.MESH)
                    pl.semaphore_signal(credit_ccw, inc=1, device_id=(right,),
                                        device_id_type=pl.DeviceIdType.MESH)

    return pl.pallas_call(
        body,
        out_shape=jax.ShapeDtypeStruct((M, N_TOT), jnp.float32),
        in_specs=[
            pl.BlockSpec(memory_space=pltpu.VMEM),
            pl.BlockSpec(memory_space=pltpu.VMEM),
            pl.BlockSpec(memory_space=pltpu.SMEM),
            pl.BlockSpec(memory_space=pltpu.SMEM),
        ],
        out_specs=pl.BlockSpec(memory_space=pl.ANY),
        scratch_shapes=[
            pltpu.VMEM((CHUNK_M, HALF_N), jnp.float32),
            pltpu.VMEM((CHUNK_M, HALF_N), jnp.float32),
            pltpu.VMEM((CHUNK_M, HALF_N), jnp.float32),
            pltpu.VMEM((CHUNK_M, HALF_N), jnp.float32),
            pltpu.VMEM((CHUNK_M, HALF_N), jnp.float32),
            pltpu.VMEM((CHUNK_M, HALF_N), jnp.float32),
            pltpu.SemaphoreType.DMA((2,)),
            pltpu.SemaphoreType.DMA((2,)),
            pltpu.SemaphoreType.DMA((2,)),
            pltpu.SemaphoreType.REGULAR,
            pltpu.SemaphoreType.REGULAR,
        ],
        compiler_params=pltpu.CompilerParams(
            collective_id=0,
            vmem_limit_bytes=64 * 1024 * 1024,
        ),
    )(x, w_mat, scale_x, scale_w)


# device time: 1420354 ns/iter; 1.0542x vs baseline; 1.0281x over previous
import jax
import jax.numpy as jnp
from jax import lax
from jax.experimental import pallas as pl
from jax.experimental.pallas import tpu as pltpu

N_DEV = 8
M = 4096
N_TOT = 8192
CHUNK_M = M // N_DEV
HALF_N = N_TOT // 2
N_STEPS = 2 * N_DEV - 2


def kernel(x, w_mat, scale_x, scale_w):
    def body(x_ref, w_ref, sx_ref, sw_ref, out_ref,
             slots_cw, slots_ccw, pc_cw, pc_ccw,
             sems_cw, sems_ccw, copy_sems, credit_cw, credit_ccw):
        my = lax.axis_index("i")
        left = lax.rem(my + N_DEV - 1, N_DEV)
        right = lax.rem(my + 1, N_DEV)

        barrier = pltpu.get_barrier_semaphore()
        for nbr in (left, right):
            pl.semaphore_signal(barrier, inc=1, device_id=(nbr,),
                                device_id_type=pl.DeviceIdType.MESH)
        pl.semaphore_wait(barrier, 2)

        scale = sx_ref[0] * sw_ref[0]

        def partial(c, half):
            xa = x_ref[pl.ds(c * CHUNK_M, CHUNK_M), :]
            wa = w_ref[:, half * HALF_N:(half + 1) * HALF_N]
            acc = lax.dot_general(xa, wa, (((1,), (0,)), ((), ())),
                                  preferred_element_type=jnp.int32)
            return acc.astype(jnp.float32) * scale

        def store_out(slots, k, c, half):
            cp = pltpu.make_async_copy(
                slots.at[k],
                out_ref.at[pl.ds(c * CHUNK_M, CHUNK_M),
                           pl.ds(half * HALF_N, HALF_N)],
                copy_sems.at[half, k])
            cp.start()
            return cp

        slots_cw[1] = partial(my, 0)
        slots_ccw[1] = partial(my, 1)

        pending = [None, None]

        for s in range(N_STEPS):
            k = s % 2
            if s >= 1:
                pl.semaphore_wait(credit_cw, 1)
                pl.semaphore_wait(credit_ccw, 1)
            rd_cw = pltpu.make_async_remote_copy(
                src_ref=slots_cw.at[1 - k], dst_ref=slots_cw.at[k],
                send_sem=sems_cw.at[0], recv_sem=sems_cw.at[1],
                device_id=(right,), device_id_type=pl.DeviceIdType.MESH)
            rd_ccw = pltpu.make_async_remote_copy(
                src_ref=slots_ccw.at[1 - k], dst_ref=slots_ccw.at[k],
                send_sem=sems_ccw.at[0], recv_sem=sems_ccw.at[1],
                device_id=(left,), device_id_type=pl.DeviceIdType.MESH)
            rd_cw.start()
            rd_ccw.start()

            if s < N_DEV - 1:
                c_cw = lax.rem(my + 2 * N_DEV - s - 1, N_DEV)
                c_ccw = lax.rem(my + s + 1, N_DEV)
                pc_cw[...] = partial(c_cw, 0)
                pc_ccw[...] = partial(c_ccw, 1)

            rd_cw.wait_recv()
            rd_ccw.wait_recv()

            new_pending = [None, None]
            if s < N_DEV - 1:
                slots_cw[k] = slots_cw[k] + pc_cw[...]
                slots_ccw[k] = slots_ccw[k] + pc_ccw[...]
                if s == N_DEV - 2:
                    new_pending[0] = store_out(slots_cw, k,
                                               lax.rem(my + 1, N_DEV), 0)
                    new_pending[1] = store_out(slots_ccw, k,
                                               lax.rem(my + N_DEV - 1, N_DEV),
                                               1)
            else:
                t = s - (N_DEV - 1)
                new_pending[0] = store_out(slots_cw, k,
                                           lax.rem(my + N_DEV - t, N_DEV), 0)
                new_pending[1] = store_out(slots_ccw, k,
                                           lax.rem(my + t, N_DEV), 1)

            rd_cw.wait_send()
            rd_ccw.wait_send()
            if pending[0] is not None:
                pending[0].wait()
                pending[1].wait()
            pending = new_pending
            if s <= N_STEPS - 2:
                pl.semaphore_signal(credit_cw, inc=1, device_id=(left,),
                                    device_id_type=pl.DeviceIdType.MESH)
                pl.semaphore_signal(credit_ccw, inc=1, device_id=(right,),
                                    device_id_type=pl.DeviceIdType.MESH)

        if pending[0] is not None:
            pending[0].wait()
            pending[1].wait()

    return pl.pallas_call(
        body,
        out_shape=jax.ShapeDtypeStruct((M, N_TOT), jnp.float32),
        in_specs=[
            pl.BlockSpec(memory_space=pltpu.VMEM),
            pl.BlockSpec(memory_space=pltpu.VMEM),
            pl.BlockSpec(memory_space=pltpu.SMEM),
            pl.BlockSpec(memory_space=pltpu.SMEM),
        ],
        out_specs=pl.BlockSpec(memory_space=pl.ANY),
        scratch_shapes=[
            pltpu.VMEM((2, CHUNK_M, HALF_N), jnp.float32),
            pltpu.VMEM((2, CHUNK_M, HALF_N), jnp.float32),
            pltpu.VMEM((CHUNK_M, HALF_N), jnp.float32),
            pltpu.VMEM((CHUNK_M, HALF_N), jnp.float32),
            pltpu.SemaphoreType.DMA((2,)),
            pltpu.SemaphoreType.DMA((2,)),
            pltpu.SemaphoreType.DMA((2, 2)),
            pltpu.SemaphoreType.REGULAR,
            pltpu.SemaphoreType.REGULAR,
        ],
        compiler_params=pltpu.CompilerParams(
            collective_id=0,
            vmem_limit_bytes=64 * 1024 * 1024,
        ),
    )(x, w_mat, scale_x, scale_w)


# device time: 1412170 ns/iter; 1.0603x vs baseline; 1.0058x over previous
import jax
import jax.numpy as jnp
from jax import lax
from jax.experimental import pallas as pl
from jax.experimental.pallas import tpu as pltpu

N_DEV = 8
M = 4096
N_TOT = 8192
CHUNK_M = M // N_DEV
PIECE_N = N_TOT // 4
N_STEPS = 2 * N_DEV - 2
LANES = ((0, 0), (1, 2 * PIECE_N), (0, PIECE_N), (1, 3 * PIECE_N))


def kernel(x, w_mat, scale_x, scale_w):
    def body(x_ref, w_ref, sx_ref, sw_ref, out_ref,
             sl0, sl1, sl2, sl3, pc0, pc1, pc2, pc3,
             sems, copy_sems, credits):
        slots = (sl0, sl1, sl2, sl3)
        pcs = (pc0, pc1, pc2, pc3)
        my = lax.axis_index("i")
        left = lax.rem(my + N_DEV - 1, N_DEV)
        right = lax.rem(my + 1, N_DEV)
        send_to = (right, left)
        ack_to = (left, right)

        barrier = pltpu.get_barrier_semaphore()
        for nbr in (left, right):
            pl.semaphore_signal(barrier, inc=1, device_id=(nbr,),
                                device_id_type=pl.DeviceIdType.MESH)
        pl.semaphore_wait(barrier, 2)

        scale = sx_ref[0] * sw_ref[0]

        def partial(c, base):
            xa = x_ref[pl.ds(c * CHUNK_M, CHUNK_M), :]
            wa = w_ref[:, base:base + PIECE_N]
            acc = lax.dot_general(xa, wa, (((1,), (0,)), ((), ())),
                                  preferred_element_type=jnp.int32)
            return acc.astype(jnp.float32) * scale

        def store_out(li, k, c):
            cp = pltpu.make_async_copy(
                slots[li].at[k],
                out_ref.at[pl.ds(c * CHUNK_M, CHUNK_M),
                           pl.ds(LANES[li][1], PIECE_N)],
                copy_sems.at[li, k])
            cp.start()
            return cp

        def rs_chunk(dirn, s):
            if dirn == 0:
                return lax.rem(my + 2 * N_DEV - s - 1, N_DEV)
            return lax.rem(my + s + 1, N_DEV)

        def ag_chunk(dirn, t):
            if dirn == 0:
                return lax.rem(my + N_DEV - t, N_DEV)
            return lax.rem(my + t, N_DEV)

        for li, (dirn, base) in enumerate(LANES):
            slots[li][1] = partial(my, base)

        pending = [None] * 4

        for s in range(N_STEPS):
            k = s % 2
            rds = []
            for li, (dirn, base) in enumerate(LANES):
                if s >= 1:
                    pl.semaphore_wait(credits.at[li], 1)
                rd = pltpu.make_async_remote_copy(
                    src_ref=slots[li].at[1 - k], dst_ref=slots[li].at[k],
                    send_sem=sems.at[li, 0], recv_sem=sems.at[li, 1],
                    device_id=(send_to[dirn],),
                    device_id_type=pl.DeviceIdType.MESH)
                rd.start()
                rds.append(rd)

            if s < N_DEV - 1:
                for li, (dirn, base) in enumerate(LANES):
                    pcs[li][...] = partial(rs_chunk(dirn, s), base)

            new_pending = [None] * 4
            for li, (dirn, base) in enumerate(LANES):
                rds[li].wait_recv()
                if s < N_DEV - 1:
                    slots[li][k] = slots[li][k] + pcs[li][...]
                    if s == N_DEV - 2:
                        own = lax.rem(my + (1 if dirn == 0 else N_DEV - 1),
                                      N_DEV)
                        new_pending[li] = store_out(li, k, own)
                else:
                    new_pending[li] = store_out(
                        li, k, ag_chunk(dirn, s - (N_DEV - 1)))

            for li, (dirn, base) in enumerate(LANES):
                rds[li].wait_send()
                if pending[li] is not None:
                    pending[li].wait()
                if s <= N_STEPS - 2:
                    pl.semaphore_signal(credits.at[li], inc=1,
                                        device_id=(ack_to[dirn],),
                                        device_id_type=pl.DeviceIdType.MESH)
            pending = new_pending

        for cp in pending:
            cp.wait()

    slot_shape = pltpu.VMEM((2, CHUNK_M, PIECE_N), jnp.float32)
    pc_shape = pltpu.VMEM((CHUNK_M, PIECE_N), jnp.float32)
    return pl.pallas_call(
        body,
        out_shape=jax.ShapeDtypeStruct((M, N_TOT), jnp.float32),
        in_specs=[
            pl.BlockSpec(memory_space=pltpu.VMEM),
            pl.BlockSpec(memory_space=pltpu.VMEM),
            pl.BlockSpec(memory_space=pltpu.SMEM),
            pl.BlockSpec(memory_space=pltpu.SMEM),
        ],
        out_specs=pl.BlockSpec(memory_space=pl.ANY),
        scratch_shapes=[
            slot_shape, slot_shape, slot_shape, slot_shape,
            pc_shape, pc_shape, pc_shape, pc_shape,
            pltpu.SemaphoreType.DMA((4, 2)),
            pltpu.SemaphoreType.DMA((4, 2)),
            pltpu.SemaphoreType.REGULAR((4,)),
        ],
        compiler_params=pltpu.CompilerParams(
            collective_id=0,
            vmem_limit_bytes=64 * 1024 * 1024,
        ),
    )(x, w_mat, scale_x, scale_w)


# device time: 1363155 ns/iter; 1.0984x vs baseline; 1.0360x over previous
import jax
import jax.numpy as jnp
from jax import lax
from jax.experimental import pallas as pl
from jax.experimental.pallas import tpu as pltpu

N_DEV = 8
M = 4096
N_TOT = 8192
CHUNK_M = M // N_DEV
PIECE_N = N_TOT // 4
N_STEPS = 2 * N_DEV - 2
LANES = ((0, 0), (1, 2 * PIECE_N), (0, PIECE_N), (1, 3 * PIECE_N))


def kernel(x, w_mat, scale_x, scale_w):
    def body(x_ref, w_ref, sx_ref, sw_ref, out_ref,
             sl0, sl1, sl2, sl3, pc0, pc1, pc2, pc3,
             sems, copy_sems, credits):
        slots = (sl0, sl1, sl2, sl3)
        pcs = (pc0, pc1, pc2, pc3)
        my = lax.axis_index("i")
        left = lax.rem(my + N_DEV - 1, N_DEV)
        right = lax.rem(my + 1, N_DEV)
        send_to = (right, left)
        ack_to = (left, right)

        barrier = pltpu.get_barrier_semaphore()
        for nbr in (left, right):
            pl.semaphore_signal(barrier, inc=1, device_id=(nbr,),
                                device_id_type=pl.DeviceIdType.MESH)
        pl.semaphore_wait(barrier, 2)

        scale = sx_ref[0] * sw_ref[0]

        def partial(c, base):
            xa = x_ref[pl.ds(c * CHUNK_M, CHUNK_M), :]
            wa = w_ref[:, base:base + PIECE_N]
            acc = lax.dot_general(xa, wa, (((1,), (0,)), ((), ())),
                                  preferred_element_type=jnp.int32)
            return acc.astype(jnp.float32) * scale

        def store_out(li, k, c):
            cp = pltpu.make_async_copy(
                slots[li].at[k],
                out_ref.at[pl.ds(c * CHUNK_M, CHUNK_M),
                           pl.ds(LANES[li][1], PIECE_N)],
                copy_sems.at[li, k])
            cp.start()
            return cp

        def rs_chunk(dirn, s):
            if dirn == 0:
                return lax.rem(my + 2 * N_DEV - s - 1, N_DEV)
            return lax.rem(my + s + 1, N_DEV)

        def ag_chunk(dirn, t):
            if dirn == 0:
                return lax.rem(my + N_DEV - t, N_DEV)
            return lax.rem(my + t, N_DEV)

        for li, (dirn, base) in enumerate(LANES):
            slots[li][1] = partial(my, base)

        def make_rd(li, s):
            k = s % 2
            return pltpu.make_async_remote_copy(
                src_ref=slots[li].at[1 - k], dst_ref=slots[li].at[k],
                send_sem=sems.at[li, 0], recv_sem=sems.at[li, 1],
                device_id=(send_to[LANES[li][0]],),
                device_id_type=pl.DeviceIdType.MESH)

        rds = [None] * 4
        pend = [[None, None] for _ in range(4)]

        for s in range(N_STEPS + 1):
            k = s % 2
            for li, (dirn, base) in enumerate(LANES):
                d = s - 1
                if s >= 1:
                    rds[li].wait_recv()
                    if d < N_DEV - 1:
                        slots[li][1 - k] = slots[li][1 - k] + pcs[li][...]
                        if d == N_DEV - 2:
                            own = lax.rem(
                                my + (1 if dirn == 0 else N_DEV - 1), N_DEV)
                            pend[li][1 - k] = store_out(li, 1 - k, own)
                    else:
                        pend[li][1 - k] = store_out(
                            li, 1 - k, ag_chunk(dirn, d - (N_DEV - 1)))
                    rds[li].wait_send()
                    if pend[li][k] is not None:
                        pend[li][k].wait()
                        pend[li][k] = None
                    if s < N_STEPS:
                        pl.semaphore_signal(credits.at[li], inc=1,
                                            device_id=(ack_to[dirn],),
                                            device_id_type=pl.DeviceIdType.MESH)
                if s < N_STEPS:
                    if s >= 1:
                        pl.semaphore_wait(credits.at[li], 1)
                    rds[li] = make_rd(li, s)
                    rds[li].start()
            if s < N_DEV - 1:
                for li, (dirn, base) in enumerate(LANES):
                    pcs[li][...] = partial(rs_chunk(dirn, s), base)

        for li in range(4):
            if pend[li][1] is not None:
                pend[li][1].wait()

    slot_shape = pltpu.VMEM((2, CHUNK_M, PIECE_N), jnp.float32)
    pc_shape = pltpu.VMEM((CHUNK_M, PIECE_N), jnp.float32)
    return pl.pallas_call(
        body,
        out_shape=jax.ShapeDtypeStruct((M, N_TOT), jnp.float32),
        in_specs=[
            pl.BlockSpec(memory_space=pltpu.VMEM),
            pl.BlockSpec(memory_space=pltpu.VMEM),
            pl.BlockSpec(memory_space=pltpu.SMEM),
            pl.BlockSpec(memory_space=pltpu.SMEM),
        ],
        out_specs=pl.BlockSpec(memory_space=pl.ANY),
        scratch_shapes=[
            slot_shape, slot_shape, slot_shape, slot_shape,
            pc_shape, pc_shape, pc_shape, pc_shape,
            pltpu.SemaphoreType.DMA((4, 2)),
            pltpu.SemaphoreType.DMA((4, 2)),
            pltpu.SemaphoreType.REGULAR((4,)),
        ],
        compiler_params=pltpu.CompilerParams(
            collective_id=0,
            vmem_limit_bytes=64 * 1024 * 1024,
        ),
    )(x, w_mat, scale_x, scale_w)


# device time: 1362388 ns/iter; 1.0991x vs baseline; 1.0006x over previous
import jax
import jax.numpy as jnp
from jax import lax
from jax.experimental import pallas as pl
from jax.experimental.pallas import tpu as pltpu

N_DEV = 8
M = 4096
N_TOT = 8192
CHUNK_M = M // N_DEV
N_LANES = 8
PIECE_N = N_TOT // N_LANES
N_STEPS = 2 * N_DEV - 2
LANES = tuple(
    (dirn, dirn * (N_TOT // 2) + p * PIECE_N)
    for p in range(N_LANES // 2) for dirn in (0, 1)
)


def kernel(x, w_mat, scale_x, scale_w):
    def body(x_ref, w_ref, sx_ref, sw_ref, out_ref, *scratch):
        slots = scratch[:N_LANES]
        pcs = scratch[N_LANES:2 * N_LANES]
        sems, copy_sems, credits = scratch[2 * N_LANES:]
        my = lax.axis_index("i")
        left = lax.rem(my + N_DEV - 1, N_DEV)
        right = lax.rem(my + 1, N_DEV)
        send_to = (right, left)
        ack_to = (left, right)

        barrier = pltpu.get_barrier_semaphore()
        for nbr in (left, right):
            pl.semaphore_signal(barrier, inc=1, device_id=(nbr,),
                                device_id_type=pl.DeviceIdType.MESH)
        pl.semaphore_wait(barrier, 2)

        scale = sx_ref[0] * sw_ref[0]

        def partial(c, base):
            xa = x_ref[pl.ds(c * CHUNK_M, CHUNK_M), :]
            wa = w_ref[:, base:base + PIECE_N]
            acc = lax.dot_general(xa, wa, (((1,), (0,)), ((), ())),
                                  preferred_element_type=jnp.int32)
            return acc.astype(jnp.float32) * scale

        def store_out(li, k, c):
            cp = pltpu.make_async_copy(
                slots[li].at[k],
                out_ref.at[pl.ds(c * CHUNK_M, CHUNK_M),
                           pl.ds(LANES[li][1], PIECE_N)],
                copy_sems.at[li, k])
            cp.start()
            return cp

        def rs_chunk(dirn, s):
            if dirn == 0:
                return lax.rem(my + 2 * N_DEV - s - 1, N_DEV)
            return lax.rem(my + s + 1, N_DEV)

        def ag_chunk(dirn, t):
            if dirn == 0:
                return lax.rem(my + N_DEV - t, N_DEV)
            return lax.rem(my + t, N_DEV)

        for li, (dirn, base) in enumerate(LANES):
            slots[li][1] = partial(my, base)

        def make_rd(li, s):
            k = s % 2
            return pltpu.make_async_remote_copy(
                src_ref=slots[li].at[1 - k], dst_ref=slots[li].at[k],
                send_sem=sems.at[li, 0], recv_sem=sems.at[li, 1],
                device_id=(send_to[LANES[li][0]],),
                device_id_type=pl.DeviceIdType.MESH)

        rds = [None] * N_LANES
        pend = [[None, None] for _ in range(N_LANES)]

        for s in range(N_STEPS + 1):
            k = s % 2
            for li, (dirn, base) in enumerate(LANES):
                d = s - 1
                if s >= 1:
                    rds[li].wait_recv()
                    if d < N_DEV - 1:
                        slots[li][1 - k] = slots[li][1 - k] + pcs[li][...]
                        if d == N_DEV - 2:
                            own = lax.rem(
                                my + (1 if dirn == 0 else N_DEV - 1), N_DEV)
                            pend[li][1 - k] = store_out(li, 1 - k, own)
                    else:
                        pend[li][1 - k] = store_out(
                            li, 1 - k, ag_chunk(dirn, d - (N_DEV - 1)))
                    rds[li].wait_send()
                    if pend[li][k] is not None:
                        pend[li][k].wait()
                        pend[li][k] = None
                    if s < N_STEPS:
                        pl.semaphore_signal(credits.at[li], inc=1,
                                            device_id=(ack_to[dirn],),
                                            device_id_type=pl.DeviceIdType.MESH)
                if s < N_STEPS:
                    if s >= 1:
                        pl.semaphore_wait(credits.at[li], 1)
                    rds[li] = make_rd(li, s)
                    rds[li].start()
            if s < N_DEV - 1:
                for li, (dirn, base) in enumerate(LANES):
                    pcs[li][...] = partial(rs_chunk(dirn, s), base)

        for li in range(N_LANES):
            if pend[li][1] is not None:
                pend[li][1].wait()

    slot_shape = pltpu.VMEM((2, CHUNK_M, PIECE_N), jnp.float32)
    pc_shape = pltpu.VMEM((CHUNK_M, PIECE_N), jnp.float32)
    return pl.pallas_call(
        body,
        out_shape=jax.ShapeDtypeStruct((M, N_TOT), jnp.float32),
        in_specs=[
            pl.BlockSpec(memory_space=pltpu.VMEM),
            pl.BlockSpec(memory_space=pltpu.VMEM),
            pl.BlockSpec(memory_space=pltpu.SMEM),
            pl.BlockSpec(memory_space=pltpu.SMEM),
        ],
        out_specs=pl.BlockSpec(memory_space=pl.ANY),
        scratch_shapes=(
            [slot_shape] * N_LANES + [pc_shape] * N_LANES + [
                pltpu.SemaphoreType.DMA((N_LANES, 2)),
                pltpu.SemaphoreType.DMA((N_LANES, 2)),
                pltpu.SemaphoreType.REGULAR((N_LANES,)),
            ]
        ),
        compiler_params=pltpu.CompilerParams(
            collective_id=0,
            vmem_limit_bytes=64 * 1024 * 1024,
        ),
    )(x, w_mat, scale_x, scale_w)
